# baseline (speedup 1.0000x reference)
import functools

import jax
import jax.numpy as jnp
from jax import lax
from jax.experimental import pallas as pl
from jax.experimental.pallas import tpu as pltpu
from jax.experimental.pallas import tpu_sc as plsc

N_ATOMS = 524288
N_MOL = 8192
FEAT = 64
MAX_Z = 100

NC = 2
NS = 16
NW = NC * NS
NCHUNK = 2
CATOMS = N_ATOMS // NCHUNK
CHUNK = CATOMS // NW
BLK = 256
NBLK = CHUNK // BLK


def _table_body(emb_ref, w_ref, b_ref, out_ref):
    w1 = w_ref[...][:, :FEAT]
    acc = lax.dot_general(
        emb_ref[...], w1, (((1,), (1,)), ((), ())),
        preferred_element_type=jnp.float32)
    out_ref[...] = acc + b_ref[...]


def _fused_table(emb, w, b2d):
    return pl.pallas_call(
        _table_body,
        out_shape=jax.ShapeDtypeStruct((MAX_Z, FEAT), jnp.float32),
    )(emb, w, b2d)


_MESH = plsc.VectorSubcoreMesh(
    core_axis_name="c", subcore_axis_name="s", num_cores=NC, num_subcores=NS)

_SCRATCH = [
    pltpu.VMEM((MAX_Z * FEAT,), jnp.float32),
    pltpu.VMEM((FEAT,), jnp.float32),
    pltpu.VMEM((BLK,), jnp.int32),
    pltpu.VMEM((BLK,), jnp.int32),
    pltpu.VMEM((BLK, FEAT), jnp.float32),
    pltpu.VMEM((BLK, FEAT), jnp.float32),
    pltpu.SMEM((BLK,), jnp.int32),
    pltpu.SMEM((BLK,), jnp.int32),
    pltpu.SMEM((BLK,), jnp.float32),
    pltpu.SMEM((BLK,), jnp.float32),
    pltpu.VMEM((BLK,), jnp.float32),
    pltpu.VMEM((BLK,), jnp.float32),
    pltpu.VMEM_SHARED((N_MOL,), jnp.float32),
    pltpu.VMEM_SHARED((NS * 2, BLK), jnp.int32),
    pltpu.VMEM_SHARED((NS * 2, BLK), jnp.float32),
    pltpu.SemaphoreType.DMA,
    pltpu.SemaphoreType.DMA,
    pltpu.SemaphoreType.DMA,
    pltpu.SemaphoreType.DMA,
    pltpu.SemaphoreType.DMA,
    pltpu.SemaphoreType.DMA,
]


def _make_sc_kernel(cbase):
  @functools.partial(
      pl.kernel,
      out_type=jax.ShapeDtypeStruct((CATOMS, FEAT), jnp.float32),
      mesh=_MESH,
      scratch_types=_SCRATCH,
      compiler_params=pltpu.CompilerParams(needs_layout_passes=False),
  )
  def _sc_featurize(tbl_hbm, w_hbm, z_hbm, s_hbm, chg_hbm, out2d_hbm,
                    tbl, wvm, sv0, sv1, ob0, ob1, zsm0, zsm1, csm0, csm1,
                    cv0, cv1, chg_sp, zsp, csp,
                    semA0, semA1, semB0, semB1, semo0, semo1):
    sv = (sv0, sv1)
    ob = (ob0, ob1)
    zsm = (zsm0, zsm1)
    csm = (csm0, csm1)
    cv = (cv0, cv1)
    semA = (semA0, semA1)
    semB = (semB0, semB1)
    semo = (semo0, semo1)

    tid = lax.axis_index("s")
    wid = tid * NC + lax.axis_index("c")
    obase = wid * CHUNK
    ibase = cbase + obase

    pltpu.sync_copy(tbl_hbm, tbl)
    pltpu.sync_copy(w_hbm, wvm)

    @pl.when(tid == 0)
    def _():
        pltpu.sync_copy(chg_hbm, chg_sp)

    plsc.subcore_barrier()

    w4 = [wvm[pl.ds(16 * j, 16)] for j in range(4)]

    def issue_a(blk, b):
        row = ibase + blk * BLK
        pltpu.async_copy(z_hbm.at[pl.ds(row, BLK)], zsp.at[tid * 2 + b],
                         semA[b])
        pltpu.async_copy(s_hbm.at[pl.ds(row, BLK)], sv[b], semA[b])

    def wait_a(blk, b):
        row = ibase + blk * BLK
        pltpu.make_async_copy(z_hbm.at[pl.ds(row, BLK)],
                              zsp.at[tid * 2 + b], semA[b]).wait()
        pltpu.make_async_copy(s_hbm.at[pl.ds(row, BLK)], sv[b],
                              semA[b]).wait()

    def issue_b1(b):
        pltpu.async_copy(zsp.at[tid * 2 + b], zsm[b], semB[b])
        pltpu.async_copy(chg_sp.at[sv[b]], cv[b], semB[b])

    def wait_b1(b):
        pltpu.make_async_copy(zsp.at[tid * 2 + b], zsm[b], semB[b]).wait()
        pltpu.make_async_copy(chg_sp.at[sv[b]], cv[b], semB[b]).wait()

    def sync_b2(b):
        pltpu.sync_copy(cv[b], csp.at[tid * 2 + b])
        pltpu.sync_copy(csp.at[tid * 2 + b], csm[b])

    issue_a(0, 0)
    issue_a(1, 1)
    wait_a(0, 0)
    issue_b1(0)
    wait_b1(0)
    sync_b2(0)

    @pl.loop(0, NBLK, step=2)
    def _blocks(blk2):
        for b in range(2):
            blk = blk2 + b
            rowbase = obase + blk * BLK

            @pl.when(blk + 2 < NBLK)
            def _():
                issue_a(blk + 2, b)

            @pl.when(blk + 1 < NBLK)
            def _():
                wait_a(blk + 1, 1 - b)
                issue_b1(1 - b)

            @pl.when(blk >= 2)
            def _():
                prow = rowbase - 2 * BLK
                pltpu.make_async_copy(
                    ob[b],
                    out2d_hbm.at[pl.ds(prow, BLK)],
                    semo[b]).wait()

            zsmb = zsm[b]
            csmb = csm[b]
            obf = ob[b]

            @pl.loop(0, BLK, unroll=4)
            def _atoms(a):
                zoff = zsmb[a] * FEAT
                c_a = csmb[a]
                for j in range(4):
                    t = tbl[pl.ds(zoff + 16 * j, 16)]
                    obf[a, pl.ds(16 * j, 16)] = t + c_a * w4[j]

            pltpu.async_copy(
                obf, out2d_hbm.at[pl.ds(rowbase, BLK)], semo[b])

            @pl.when(blk + 1 < NBLK)
            def _():
                wait_b1(1 - b)
                sync_b2(1 - b)

    for b in range(2):
        tail = obase + (NBLK - 2 + b) * BLK
        pltpu.make_async_copy(
            ob[b],
            out2d_hbm.at[pl.ds(tail, BLK)],
            semo[b]).wait()

  return _sc_featurize


_SC_KERNELS = [_make_sc_kernel(k * CATOMS) for k in range(NCHUNK)]


def kernel(atomic_numbers, per_system_total_charge, atomic_subsystem_indices,
           emb_table, W, b):
    z = atomic_numbers.astype(jnp.int32)
    s = atomic_subsystem_indices.astype(jnp.int32)
    emb = emb_table.astype(jnp.float32)
    w = W.astype(jnp.float32)
    chg = per_system_total_charge.astype(jnp.float32)
    tbl = _fused_table(emb, w, b.astype(jnp.float32).reshape(1, FEAT))
    w_last = w[:, FEAT]
    tbl_flat = tbl.reshape(-1)
    parts = [k(tbl_flat, w_last, z, s, chg) for k in _SC_KERNELS]
    return jnp.concatenate(parts, axis=0)


# speedup vs baseline: 1.0795x; 1.0795x over previous
import functools

import jax
import jax.numpy as jnp
from jax import lax
from jax.experimental import pallas as pl
from jax.experimental.pallas import tpu as pltpu
from jax.experimental.pallas import tpu_sc as plsc

N_ATOMS = 524288
N_MOL = 8192
FEAT = 64
MAX_Z = 100

NC = 2
NS = 16
NW = NC * NS
NCHUNK = 4
CATOMS = N_ATOMS // NCHUNK
CHUNK = CATOMS // NW
BLK = 256
NBLK = CHUNK // BLK


def _table_body(emb_ref, w_ref, b_ref, out_ref):
    w1 = w_ref[...][:, :FEAT]
    acc = lax.dot_general(
        emb_ref[...], w1, (((1,), (1,)), ((), ())),
        preferred_element_type=jnp.float32)
    out_ref[...] = acc + b_ref[...]


def _fused_table(emb, w, b2d):
    return pl.pallas_call(
        _table_body,
        out_shape=jax.ShapeDtypeStruct((MAX_Z, FEAT), jnp.float32),
    )(emb, w, b2d)


_MESH = plsc.VectorSubcoreMesh(
    core_axis_name="c", subcore_axis_name="s", num_cores=NC, num_subcores=NS)

_SCRATCH = [
    pltpu.VMEM((MAX_Z * FEAT,), jnp.float32),
    pltpu.VMEM((FEAT,), jnp.float32),
    pltpu.VMEM((BLK,), jnp.int32),
    pltpu.VMEM((BLK,), jnp.int32),
    pltpu.VMEM((BLK, FEAT), jnp.float32),
    pltpu.VMEM((BLK, FEAT), jnp.float32),
    pltpu.SMEM((BLK,), jnp.int32),
    pltpu.SMEM((BLK,), jnp.int32),
    pltpu.SMEM((BLK,), jnp.float32),
    pltpu.SMEM((BLK,), jnp.float32),
    pltpu.VMEM((BLK,), jnp.float32),
    pltpu.VMEM((BLK,), jnp.float32),
    pltpu.VMEM_SHARED((N_MOL,), jnp.float32),
    pltpu.VMEM_SHARED((NS * 2, BLK), jnp.int32),
    pltpu.VMEM_SHARED((NS * 2, BLK), jnp.float32),
    pltpu.SemaphoreType.DMA,
    pltpu.SemaphoreType.DMA,
    pltpu.SemaphoreType.DMA,
    pltpu.SemaphoreType.DMA,
    pltpu.SemaphoreType.DMA,
    pltpu.SemaphoreType.DMA,
]


def _make_sc_kernel(cbase):
  @functools.partial(
      pl.kernel,
      out_type=jax.ShapeDtypeStruct((CATOMS, FEAT), jnp.float32),
      mesh=_MESH,
      scratch_types=_SCRATCH,
      compiler_params=pltpu.CompilerParams(needs_layout_passes=False),
  )
  def _sc_featurize(tbl_hbm, w_hbm, z_hbm, s_hbm, chg_hbm, out2d_hbm,
                    tbl, wvm, sv0, sv1, ob0, ob1, zsm0, zsm1, csm0, csm1,
                    cv0, cv1, chg_sp, zsp, csp,
                    semA0, semA1, semB0, semB1, semo0, semo1):
    sv = (sv0, sv1)
    ob = (ob0, ob1)
    zsm = (zsm0, zsm1)
    csm = (csm0, csm1)
    cv = (cv0, cv1)
    semA = (semA0, semA1)
    semB = (semB0, semB1)
    semo = (semo0, semo1)

    tid = lax.axis_index("s")
    wid = tid * NC + lax.axis_index("c")
    obase = wid * CHUNK
    ibase = cbase + obase

    pltpu.sync_copy(tbl_hbm, tbl)
    pltpu.sync_copy(w_hbm, wvm)

    @pl.when(tid == 0)
    def _():
        pltpu.sync_copy(chg_hbm, chg_sp)

    plsc.subcore_barrier()

    w4 = [wvm[pl.ds(16 * j, 16)] for j in range(4)]

    def issue_a(blk, b):
        row = ibase + blk * BLK
        pltpu.async_copy(z_hbm.at[pl.ds(row, BLK)], zsp.at[tid * 2 + b],
                         semA[b])
        pltpu.async_copy(s_hbm.at[pl.ds(row, BLK)], sv[b], semA[b])

    def wait_a(blk, b):
        row = ibase + blk * BLK
        pltpu.make_async_copy(z_hbm.at[pl.ds(row, BLK)],
                              zsp.at[tid * 2 + b], semA[b]).wait()
        pltpu.make_async_copy(s_hbm.at[pl.ds(row, BLK)], sv[b],
                              semA[b]).wait()

    def issue_b1(b):
        pltpu.async_copy(zsp.at[tid * 2 + b], zsm[b], semB[b])
        pltpu.async_copy(chg_sp.at[sv[b]], cv[b], semB[b])

    def wait_b1(b):
        pltpu.make_async_copy(zsp.at[tid * 2 + b], zsm[b], semB[b]).wait()
        pltpu.make_async_copy(chg_sp.at[sv[b]], cv[b], semB[b]).wait()

    def sync_b2(b):
        pltpu.sync_copy(cv[b], csp.at[tid * 2 + b])
        pltpu.sync_copy(csp.at[tid * 2 + b], csm[b])

    issue_a(0, 0)
    issue_a(1, 1)
    wait_a(0, 0)
    issue_b1(0)
    wait_b1(0)
    sync_b2(0)

    @pl.loop(0, NBLK, step=2)
    def _blocks(blk2):
        for b in range(2):
            blk = blk2 + b
            rowbase = obase + blk * BLK

            @pl.when(blk + 2 < NBLK)
            def _():
                issue_a(blk + 2, b)

            @pl.when(blk + 1 < NBLK)
            def _():
                wait_a(blk + 1, 1 - b)
                issue_b1(1 - b)

            @pl.when(blk >= 2)
            def _():
                prow = rowbase - 2 * BLK
                pltpu.make_async_copy(
                    ob[b],
                    out2d_hbm.at[pl.ds(prow, BLK)],
                    semo[b]).wait()

            zsmb = zsm[b]
            csmb = csm[b]
            obf = ob[b]

            @pl.loop(0, BLK, unroll=4)
            def _atoms(a):
                zoff = zsmb[a] * FEAT
                c_a = csmb[a]
                for j in range(4):
                    t = tbl[pl.ds(zoff + 16 * j, 16)]
                    obf[a, pl.ds(16 * j, 16)] = t + c_a * w4[j]

            pltpu.async_copy(
                obf, out2d_hbm.at[pl.ds(rowbase, BLK)], semo[b])

            @pl.when(blk + 1 < NBLK)
            def _():
                wait_b1(1 - b)
                sync_b2(1 - b)

    for b in range(2):
        tail = obase + (NBLK - 2 + b) * BLK
        pltpu.make_async_copy(
            ob[b],
            out2d_hbm.at[pl.ds(tail, BLK)],
            semo[b]).wait()

  return _sc_featurize


_SC_KERNELS = [_make_sc_kernel(k * CATOMS) for k in range(NCHUNK)]


def kernel(atomic_numbers, per_system_total_charge, atomic_subsystem_indices,
           emb_table, W, b):
    z = atomic_numbers.astype(jnp.int32)
    s = atomic_subsystem_indices.astype(jnp.int32)
    emb = emb_table.astype(jnp.float32)
    w = W.astype(jnp.float32)
    chg = per_system_total_charge.astype(jnp.float32)
    tbl = _fused_table(emb, w, b.astype(jnp.float32).reshape(1, FEAT))
    w_last = w[:, FEAT]
    tbl_flat = tbl.reshape(-1)
    out = jnp.zeros((N_ATOMS, FEAT), jnp.float32)
    for i, k in enumerate(_SC_KERNELS):
        part = k(tbl_flat, w_last, z, s, chg)
        out = lax.dynamic_update_slice(out, part, (i * CATOMS, 0))
    return out


# speedup vs baseline: 1.1196x; 1.0372x over previous
import functools

import jax
import jax.numpy as jnp
from jax import lax
from jax.experimental import pallas as pl
from jax.experimental.pallas import tpu as pltpu
from jax.experimental.pallas import tpu_sc as plsc

N_ATOMS = 524288
N_MOL = 8192
FEAT = 64
MAX_Z = 100
KDIM = 104
BN = 1024
NBN = N_ATOMS // BN

NC = 2
NS = 16
NW = NC * NS
CHUNK = N_ATOMS // NW
BLKG = 2048
NBLKG = CHUNK // BLKG


def _table_body(emb_ref, w_ref, wl_ref, b_ref, out_ref):
    w1 = w_ref[...][:, :FEAT]
    acc = lax.dot_general(
        emb_ref[...], w1, (((1,), (1,)), ((), ())),
        preferred_element_type=jnp.float32) + b_ref[...]
    ii = lax.broadcasted_iota(jnp.int32, (KDIM, FEAT), 0)
    out_ref[...] = jnp.where(
        ii < MAX_Z, acc, jnp.where(ii == MAX_Z, wl_ref[...], 0.0))


def _fused_table(emb104, w, wl2d, b2d):
    return pl.pallas_call(
        _table_body,
        out_shape=jax.ShapeDtypeStruct((KDIM, FEAT), jnp.float32),
    )(emb104, w, wl2d, b2d)


def _featurize_body(z_ref, c_ref, t_ref, out_ref):
    zb = z_ref[...][None, :]
    cb = c_ref[...][None, :]
    ii = lax.broadcasted_iota(jnp.int32, (KDIM, BN), 0)
    mt = jnp.where(ii == zb, 1.0, 0.0) + jnp.where(ii == MAX_Z, cb, 0.0)
    out_t = lax.dot_general(
        t_ref[...], mt, (((0,), (0,)), ((), ())),
        preferred_element_type=jnp.float32)
    out_ref[...] = out_t.T


def _featurize_tc(z, c_exp, tbl):
    return pl.pallas_call(
        _featurize_body,
        grid=(NBN,),
        in_specs=[
            pl.BlockSpec((BN,), lambda i: (i,)),
            pl.BlockSpec((BN,), lambda i: (i,)),
            pl.BlockSpec((KDIM, FEAT), lambda i: (0, 0)),
        ],
        out_specs=pl.BlockSpec((BN, FEAT), lambda i: (i, 0)),
        out_shape=jax.ShapeDtypeStruct((N_ATOMS, FEAT), jnp.float32),
    )(z, c_exp, tbl)


_MESH = plsc.VectorSubcoreMesh(
    core_axis_name="c", subcore_axis_name="s", num_cores=NC, num_subcores=NS)


@functools.partial(
    pl.kernel,
    out_type=jax.ShapeDtypeStruct((N_ATOMS,), jnp.float32),
    mesh=_MESH,
    scratch_types=[
        pltpu.VMEM((BLKG,), jnp.int32),
        pltpu.VMEM((BLKG,), jnp.int32),
        pltpu.VMEM((BLKG,), jnp.float32),
        pltpu.VMEM((BLKG,), jnp.float32),
        pltpu.VMEM_SHARED((N_MOL,), jnp.float32),
        pltpu.SemaphoreType.DMA,
        pltpu.SemaphoreType.DMA,
        pltpu.SemaphoreType.DMA,
        pltpu.SemaphoreType.DMA,
        pltpu.SemaphoreType.DMA,
        pltpu.SemaphoreType.DMA,
    ],
    compiler_params=pltpu.CompilerParams(needs_layout_passes=False),
)
def _sc_expand(s_hbm, chg_hbm, out_hbm,
               sv0, sv1, cv0, cv1, chg_sp,
               semS0, semS1, semG0, semG1, semO0, semO1):
    sv = (sv0, sv1)
    cv = (cv0, cv1)
    semS = (semS0, semS1)
    semG = (semG0, semG1)
    semO = (semO0, semO1)

    tid = lax.axis_index("s")
    wid = tid * NC + lax.axis_index("c")
    base = wid * CHUNK

    @pl.when(tid == 0)
    def _():
        pltpu.sync_copy(chg_hbm, chg_sp)

    plsc.subcore_barrier()

    def issue_s(blk, b):
        pltpu.async_copy(s_hbm.at[pl.ds(base + blk * BLKG, BLKG)], sv[b],
                         semS[b])

    def wait_s(blk, b):
        pltpu.make_async_copy(s_hbm.at[pl.ds(base + blk * BLKG, BLKG)],
                              sv[b], semS[b]).wait()

    issue_s(0, 0)
    issue_s(1, 1)

    @pl.loop(0, NBLKG, step=2)
    def _blocks(blk2):
        for b in range(2):
            blk = blk2 + b
            row = base + blk * BLKG

            wait_s(blk, b)
            pltpu.async_copy(chg_sp.at[sv[b]], cv[b], semG[b])
            pltpu.make_async_copy(chg_sp.at[sv[b]], cv[b], semG[b]).wait()

            @pl.when(blk >= 2)
            def _():
                prow = base + (blk - 2) * BLKG
                pltpu.make_async_copy(
                    cv[b], out_hbm.at[pl.ds(prow, BLKG)], semO[b]).wait()

            pltpu.async_copy(cv[b], out_hbm.at[pl.ds(row, BLKG)], semO[b])

            @pl.when(blk + 2 < NBLKG)
            def _():
                issue_s(blk + 2, b)

    for b in range(2):
        tail = base + (NBLKG - 2 + b) * BLKG
        pltpu.make_async_copy(
            cv[b], out_hbm.at[pl.ds(tail, BLKG)], semO[b]).wait()


def kernel(atomic_numbers, per_system_total_charge, atomic_subsystem_indices,
           emb_table, W, b):
    z = atomic_numbers.astype(jnp.int32)
    s = atomic_subsystem_indices.astype(jnp.int32)
    emb = emb_table.astype(jnp.float32)
    w = W.astype(jnp.float32)
    chg = per_system_total_charge.astype(jnp.float32)

    emb104 = jnp.pad(emb, ((0, KDIM - MAX_Z), (0, 0)))
    wl2d = w[:, FEAT].reshape(1, FEAT)
    tbl = _fused_table(emb104, w, wl2d, b.astype(jnp.float32).reshape(1, FEAT))

    c_exp = _sc_expand(s, chg)
    return _featurize_tc(z, c_exp, tbl)
